# baseline (device time: 20471 ns/iter reference)
import jax
import jax.numpy as jnp
from jax import lax
from jax.experimental import pallas as pl
from jax.experimental.pallas import tpu as pltpu

N_DEV = 4
NQ = 8


def kernel(A, B):
    m, k = A.shape
    _, n = B.shape
    mq = m // NQ

    def body(a_ref, b_ref, out_ref, comm_ref, send_sems, recv_sems):
        my_pos = lax.axis_index("i")
        y_partner = my_pos ^ 1
        x_partner = 3 - my_pos

        nh = NQ // 2
        partner1 = [y_partner] * nh + [x_partner] * nh
        partner2 = [x_partner] * nh + [y_partner] * nh
        order = [q for pair in zip(range(nh), range(nh, NQ)) for q in pair]

        barrier_sem = pltpu.get_barrier_semaphore()
        for nbr in [y_partner, x_partner]:
            pl.semaphore_signal(
                barrier_sem, inc=1,
                device_id=(nbr,), device_id_type=pl.DeviceIdType.MESH,
            )
        pl.semaphore_wait(barrier_sem, 2)

        def rdma(q, stage, partner):
            return pltpu.make_async_remote_copy(
                src_ref=out_ref.at[pl.ds(q * mq, mq)],
                dst_ref=comm_ref.at[stage, q],
                send_sem=send_sems.at[stage, q],
                recv_sem=recv_sems.at[stage, q],
                device_id=(partner,),
                device_id_type=pl.DeviceIdType.MESH,
            )

        stage1 = []
        for q in range(NQ):
            out_ref[q * mq:(q + 1) * mq, :] = jnp.dot(
                a_ref[q * mq:(q + 1) * mq, :], b_ref[:, :],
                preferred_element_type=jnp.float32)
            s = rdma(q, 0, partner1[q])
            s.start()
            stage1.append(s)

        stage2 = [None] * NQ
        for q in order:
            stage1[q].wait_recv()
            stage1[q].wait_send()
            out_ref[q * mq:(q + 1) * mq, :] += comm_ref[0, q, :, :]
            s = rdma(q, 1, partner2[q])
            s.start()
            stage2[q] = s

        for q in order:
            stage2[q].wait_recv()
            stage2[q].wait_send()
            z = out_ref[q * mq:(q + 1) * mq, :] + comm_ref[1, q, :, :]
            out_ref[q * mq:(q + 1) * mq, :] = z / (1.0 + jnp.exp(-z))

    return pl.pallas_call(
        body,
        out_shape=jax.ShapeDtypeStruct((m, n), jnp.float32),
        in_specs=[
            pl.BlockSpec(memory_space=pltpu.VMEM),
            pl.BlockSpec(memory_space=pltpu.VMEM),
        ],
        out_specs=pl.BlockSpec(memory_space=pltpu.VMEM),
        scratch_shapes=[
            pltpu.VMEM((2, NQ, mq, n), jnp.float32),
            pltpu.SemaphoreType.DMA((2, NQ)),
            pltpu.SemaphoreType.DMA((2, NQ)),
        ],
        compiler_params=pltpu.CompilerParams(collective_id=0),
    )(A, B)
